# baseline (speedup 1.0000x reference)
import functools

import jax
import jax.numpy as jnp
from jax import lax
from jax.experimental import pallas as pl
from jax.experimental.pallas import tpu as pltpu
from jax.experimental.pallas import tpu_sc as plsc

NC = 2
NS = 16
L = 16
NW = NC * NS
CH = 512
CHLAST = 576
CSTRIDE = 640
MMAX = 768
RROWS = 32
GCH = 8
GCAP = 144
NG = 8


def _extract(user, item, utab_t, itab_t, utail, itail):
    B = user.shape[0]
    D, V = utab_t.shape
    TAIL = V % CH
    VMAIN = V - TAIL
    nch_total = VMAIN // CH
    per_w = nch_total // NW
    extra = nch_total - per_w * NW

    mesh = plsc.VectorSubcoreMesh(
        core_axis_name="c", subcore_axis_name="s", num_cores=NC, num_subcores=NS
    )

    @functools.partial(
        pl.kernel,
        out_type=(jax.ShapeDtypeStruct((B * D,), jnp.float32),
                  jax.ShapeDtypeStruct((B * D,), jnp.float32)),
        mesh=mesh,
        compiler_params=pltpu.CompilerParams(needs_layout_passes=False),
        cost_estimate=pl.CostEstimate(flops=0, transcendentals=0,
                                      bytes_accessed=2 * 4 * D * V),
        scratch_types=[
            pltpu.VMEM((B,), jnp.int32),
            pltpu.VMEM((D, CH), jnp.float32),
            pltpu.VMEM((D, CH), jnp.float32),
            pltpu.VMEM((D, TAIL), jnp.float32),
            pltpu.VMEM((MMAX + L,), jnp.int32),
            pltpu.VMEM((MMAX + L,), jnp.int32),
            pltpu.VMEM((MMAX + L,), jnp.int32),
            pltpu.VMEM((MMAX + L,), jnp.int32),
            pltpu.VMEM((NG * GCAP,), jnp.int32),
            pltpu.VMEM((NG * GCAP,), jnp.int32),
            pltpu.VMEM((RROWS * 128,), jnp.float32),
            pltpu.SMEM((4 + NG,), jnp.int32),
            [pltpu.SemaphoreType.DMA] * 3,
        ],
    )
    def extract(user_hbm, item_hbm, utab_hbm, itab_hbm, utail_hbm, itail_hbm,
                ou_hbm, oi_hbm, idx_v, chunk_a, chunk_b, tail_v, mu_v, mb_v,
                cu_v, cb_v, bu_v, bb_v, rows_v, cnt_s, sems):
        wid = lax.axis_index("s") * NC + lax.axis_index("c")
        iota = lax.iota(jnp.int32, L)
        c_start = wid * per_w + lax.min(wid, extra)
        nch = per_w + jnp.where(wid < extra, 1, 0)

        def phase(idx_hbm, tab_hbm, tail_hbm, out_hbm):
            pltpu.sync_copy(idx_hbm, idx_v)
            pltpu.sync_copy(tail_hbm, tail_v)
            slab_lo = c_start * CH
            slab_hi = jnp.where(c_start + nch >= nch_total, V,
                                (c_start + nch) * CH)

            def scan(bi, ptr):
                v16 = idx_v[pl.ds(bi * L, L)]
                m = (v16 >= slab_lo) & (v16 < slab_hi)
                plsc.store_compressed(mu_v.at[pl.ds(ptr, L)], v16, mask=m)
                plsc.store_compressed(mb_v.at[pl.ds(ptr, L)], bi * L + iota,
                                      mask=m)
                return ptr + plsc.all_reduce_population_count(m)[0]

            num = lax.fori_loop(0, B // L, scan, 0)
            mu_v[pl.ds(num, L)] = jnp.full((L,), -1, jnp.int32)

            nblk = (num + L - 1) // L
            for g in range(NG):
                g_lo = slab_lo + g * (GCH * CH)
                g_hi = slab_lo + (g + 1) * (GCH * CH)

                def bscan(bi, ptr, g=g, g_lo=g_lo, g_hi=g_hi):
                    v16 = mu_v[pl.ds(bi * L, L)]
                    b16 = mb_v[pl.ds(bi * L, L)]
                    m = (v16 >= g_lo) & (v16 < g_hi)
                    plsc.store_compressed(bu_v.at[pl.ds(g * GCAP + ptr, L)],
                                          v16, mask=m)
                    plsc.store_compressed(bb_v.at[pl.ds(g * GCAP + ptr, L)],
                                          b16, mask=m)
                    return ptr + plsc.all_reduce_population_count(m)[0]

                ng = lax.fori_loop(0, nblk, bscan, 0)
                bu_v[pl.ds(g * GCAP + ng, L)] = jnp.full((L,), -1, jnp.int32)
                cnt_s[4 + g] = ng

            def cscan_range(lo, hi):
                def cscan(bi, ptr):
                    v16 = mu_v[pl.ds(bi * L, L)]
                    b16 = mb_v[pl.ds(bi * L, L)]
                    m = (v16 >= lo) & (v16 < hi)
                    plsc.store_compressed(cu_v.at[pl.ds(ptr, L)], v16, mask=m)
                    plsc.store_compressed(cb_v.at[pl.ds(ptr, L)], b16, mask=m)
                    return ptr + plsc.all_reduce_population_count(m)[0]

                return lax.fori_loop(0, (num + L - 1) // L, cscan, 0)

            def emit_from(src_v, base_off, nc_):
                def emit(mi, rcnt):
                    ul = cu_v[pl.ds(mi, L)][0] - base_off
                    b = cb_v[pl.ds(mi, L)][0]
                    slot = rcnt & (RROWS - 1)

                    @pl.when(rcnt >= RROWS)
                    def _():
                        pltpu.make_async_copy(
                            tab_hbm.at[0, pl.ds(0, D)],
                            rows_v.at[pl.ds(0, D)], sems[2]).wait()

                    roff = pl.multiple_of(slot * 128, 128)
                    ulv = jnp.full((L,), ul, jnp.int32)
                    for q in range(D // L):
                        rows_v[pl.ds(roff + q * L, L)] = plsc.load_gather(
                            src_v, [q * L + iota, ulv])
                    pltpu.async_copy(rows_v.at[pl.ds(roff, D)],
                                     out_hbm.at[pl.ds(b * D, D)], sems[2])
                    return rcnt + 1

                cnt_s[0] = lax.fori_loop(0, nc_, emit, cnt_s[0])

            def fire(c, buf, sem):
                ubase = pl.multiple_of(c * CH, 128)
                pltpu.async_copy(tab_hbm.at[:, pl.ds(ubase, CH)], buf, sem)

            def consume(c, buf, sem):
                pltpu.make_async_copy(tab_hbm.at[:, pl.ds(0, CH)], buf,
                                      sem).wait()
                ubase = c * CH
                g = (c - c_start) // GCH
                goff = g * GCAP
                ng = cnt_s[4 + g]

                def cscan(bi, ptr):
                    v16 = bu_v[pl.ds(goff + bi * L, L)]
                    b16 = bb_v[pl.ds(goff + bi * L, L)]
                    m = (v16 >= ubase) & (v16 < ubase + CH)
                    plsc.store_compressed(cu_v.at[pl.ds(ptr, L)], v16, mask=m)
                    plsc.store_compressed(cb_v.at[pl.ds(ptr, L)], b16, mask=m)
                    return ptr + plsc.all_reduce_population_count(m)[0]

                nc_ = lax.fori_loop(0, (ng + L - 1) // L, cscan, 0)
                emit_from(buf, ubase, nc_)

            cnt_s[0] = 0
            fire(c_start, chunk_a, sems[0])

            def pair_body(k, _):
                c = c_start + 2 * k

                @pl.when(2 * k + 1 < nch)
                def _():
                    fire(c + 1, chunk_b, sems[1])

                consume(c, chunk_a, sems[0])

                @pl.when(2 * k + 2 < nch)
                def _():
                    fire(c + 2, chunk_a, sems[0])

                @pl.when(2 * k + 1 < nch)
                def _():
                    consume(c + 1, chunk_b, sems[1])

                return 0

            lax.fori_loop(0, (nch + 1) // 2, pair_body, 0)

            nt = cscan_range(VMAIN, V)
            emit_from(tail_v, VMAIN, nt)

            nrows = cnt_s[0]

            def rdrain(k, _):
                pltpu.make_async_copy(tab_hbm.at[0, pl.ds(0, D)],
                                      rows_v.at[pl.ds(0, D)], sems[2]).wait()
                return 0

            lax.fori_loop(0, lax.min(nrows, RROWS), rdrain, 0)

        phase(user_hbm, utab_hbm, utail_hbm, ou_hbm)
        phase(item_hbm, itab_hbm, itail_hbm, oi_hbm)

    return extract(user, item, utab_t, itab_t, utail, itail)


def _join(user, item, uemb, iemb, ubias, ibias):
    B = user.shape[0]
    D = uemb.shape[0] // B
    bpw = B // NW

    mesh = plsc.VectorSubcoreMesh(
        core_axis_name="c", subcore_axis_name="s", num_cores=NC, num_subcores=NS
    )

    @functools.partial(
        pl.kernel,
        out_type=jax.ShapeDtypeStruct((B * D,), jnp.float32),
        mesh=mesh,
        scratch_types=[
            pltpu.VMEM((bpw,), jnp.int32),
            pltpu.VMEM((bpw,), jnp.int32),
            pltpu.VMEM((bpw * D,), jnp.float32),
            pltpu.VMEM((bpw * D,), jnp.float32),
            pltpu.VMEM((bpw,), jnp.float32),
            pltpu.VMEM((bpw,), jnp.float32),
            pltpu.VMEM((bpw * D,), jnp.float32),
            [pltpu.SemaphoreType.DMA] * 4,
        ],
    )
    def join(user_hbm, item_hbm, ue_hbm, ie_hbm, ub_hbm, ib_hbm, out_hbm,
             uidx_v, iidx_v, ue_v, ie_v, ub_v, ib_v, o_v, sems):
        wid = lax.axis_index("s") * NC + lax.axis_index("c")
        base = wid * bpw

        pltpu.sync_copy(user_hbm.at[pl.ds(base, bpw)], uidx_v)
        pltpu.sync_copy(item_hbm.at[pl.ds(base, bpw)], iidx_v)

        cub = pltpu.async_copy(ub_hbm.at[uidx_v], ub_v, sems[0])
        cib = pltpu.async_copy(ib_hbm.at[iidx_v], ib_v, sems[1])
        cue = pltpu.async_copy(ue_hbm.at[pl.ds(base * D, bpw * D)], ue_v,
                               sems[2])
        cie = pltpu.async_copy(ie_hbm.at[pl.ds(base * D, bpw * D)], ie_v,
                               sems[3])
        cub.wait()
        cib.wait()
        cue.wait()
        cie.wait()

        def blk(bi, _):
            b0 = bi * L
            ub16 = ub_v[pl.ds(b0, L)]
            ib16 = ib_v[pl.ds(b0, L)]
            for j in range(L):
                r = (b0 + j) * D
                ubb = jnp.full((L,), ub16[j])
                ibb = jnp.full((L,), ib16[j])
                for q in range(D // L):
                    sl = pl.ds(r + q * L, L)
                    o_v[sl] = (ue_v[sl] + ubb) * (ie_v[sl] + ibb)
            return 0

        lax.fori_loop(0, bpw // L, blk, 0)

        pltpu.sync_copy(o_v, out_hbm.at[pl.ds(base * D, bpw * D)])

    return join(user, item, uemb, iemb, ubias, ibias)


def kernel(user, item, user_table, item_table, user_bias, item_bias):
    B = user.shape[0]
    D = user_table.shape[1]
    V = user_table.shape[0]
    vmain = V - (V % 512)
    uemb, iemb = _extract(user, item, user_table.T, item_table.T,
                          user_table[vmain:].T, item_table[vmain:].T)
    out = _join(user, item, uemb, iemb, user_bias[:, 0], item_bias[:, 0])
    return out.reshape(B, D)


# speedup vs baseline: 1.0421x; 1.0421x over previous
import functools

import jax
import jax.numpy as jnp
from jax import lax
from jax.experimental import pallas as pl
from jax.experimental.pallas import tpu as pltpu
from jax.experimental.pallas import tpu_sc as plsc

NC = 2
NS = 16
L = 16
NW = NC * NS
CH = 512
CHLAST = 576
CSTRIDE = 640
MMAX = 768
RROWS = 32


def _extract(user, item, utab_t, itab_t, utail, itail):
    B = user.shape[0]
    D, V = utab_t.shape
    TAIL = V % CH
    VMAIN = V - TAIL
    nch_total = VMAIN // CH
    per_w = nch_total // NW
    extra = nch_total - per_w * NW

    mesh = plsc.VectorSubcoreMesh(
        core_axis_name="c", subcore_axis_name="s", num_cores=NC, num_subcores=NS
    )

    @functools.partial(
        pl.kernel,
        out_type=(jax.ShapeDtypeStruct((B * D,), jnp.float32),
                  jax.ShapeDtypeStruct((B * D,), jnp.float32)),
        mesh=mesh,
        compiler_params=pltpu.CompilerParams(needs_layout_passes=False),
        cost_estimate=pl.CostEstimate(flops=0, transcendentals=0,
                                      bytes_accessed=2 * 4 * D * V),
        scratch_types=[
            pltpu.VMEM((B,), jnp.int32),
            pltpu.VMEM((D, CH), jnp.float32),
            pltpu.VMEM((D, CH), jnp.float32),
            pltpu.VMEM((D, TAIL), jnp.float32),
            pltpu.VMEM((MMAX + L,), jnp.int32),
            pltpu.VMEM((MMAX + L,), jnp.int32),
            pltpu.VMEM((MMAX + L,), jnp.int32),
            pltpu.VMEM((MMAX + L,), jnp.int32),
            pltpu.VMEM((RROWS * 128,), jnp.float32),
            pltpu.SMEM((4,), jnp.int32),
            [pltpu.SemaphoreType.DMA] * 3,
        ],
    )
    def extract(user_hbm, item_hbm, utab_hbm, itab_hbm, utail_hbm, itail_hbm,
                ou_hbm, oi_hbm, idx_v, chunk_a, chunk_b, tail_v, mu_v, mb_v,
                cu_v, cb_v, rows_v, cnt_s, sems):
        wid = lax.axis_index("s") * NC + lax.axis_index("c")
        iota = lax.iota(jnp.int32, L)
        c_start = wid * per_w + lax.min(wid, extra)
        nch = per_w + jnp.where(wid < extra, 1, 0)

        def phase(idx_hbm, tab_hbm, tail_hbm, out_hbm):
            pltpu.sync_copy(idx_hbm, idx_v)
            pltpu.sync_copy(tail_hbm, tail_v)
            slab_lo = c_start * CH
            slab_hi = jnp.where(c_start + nch >= nch_total, V,
                                (c_start + nch) * CH)

            def scan(bi, ptr):
                v16 = idx_v[pl.ds(bi * L, L)]
                m = (v16 >= slab_lo) & (v16 < slab_hi)
                plsc.store_compressed(mu_v.at[pl.ds(ptr, L)], v16, mask=m)
                plsc.store_compressed(mb_v.at[pl.ds(ptr, L)], bi * L + iota,
                                      mask=m)
                return ptr + plsc.all_reduce_population_count(m)[0]

            num = lax.fori_loop(0, B // L, scan, 0)
            mu_v[pl.ds(num, L)] = jnp.full((L,), -1, jnp.int32)

            def cscan_range(lo, hi):
                def cscan(bi, ptr):
                    v16 = mu_v[pl.ds(bi * L, L)]
                    b16 = mb_v[pl.ds(bi * L, L)]
                    m = (v16 >= lo) & (v16 < hi)
                    plsc.store_compressed(cu_v.at[pl.ds(ptr, L)], v16, mask=m)
                    plsc.store_compressed(cb_v.at[pl.ds(ptr, L)], b16, mask=m)
                    return ptr + plsc.all_reduce_population_count(m)[0]

                return lax.fori_loop(0, (num + L - 1) // L, cscan, 0)

            def emit_from(src_v, base_off, nc_):
                def emit(mi, rcnt):
                    ul = cu_v[pl.ds(mi, L)][0] - base_off
                    b = cb_v[pl.ds(mi, L)][0]
                    slot = rcnt & (RROWS - 1)

                    @pl.when(rcnt >= RROWS)
                    def _():
                        pltpu.make_async_copy(
                            tab_hbm.at[0, pl.ds(0, D)],
                            rows_v.at[pl.ds(0, D)], sems[2]).wait()

                    roff = pl.multiple_of(slot * 128, 128)
                    ulv = jnp.full((L,), ul, jnp.int32)
                    for q in range(D // L):
                        rows_v[pl.ds(roff + q * L, L)] = plsc.load_gather(
                            src_v, [q * L + iota, ulv])
                    pltpu.async_copy(rows_v.at[pl.ds(roff, D)],
                                     out_hbm.at[pl.ds(b * D, D)], sems[2])
                    return rcnt + 1

                cnt_s[0] = lax.fori_loop(0, nc_, emit, cnt_s[0])

            def fire(c, buf, sem):
                ubase = pl.multiple_of(c * CH, 128)
                pltpu.async_copy(tab_hbm.at[:, pl.ds(ubase, CH)], buf, sem)

            def consume(c, buf, sem):
                pltpu.make_async_copy(tab_hbm.at[:, pl.ds(0, CH)], buf,
                                      sem).wait()
                ubase = c * CH
                nc_ = cscan_range(ubase, ubase + CH)
                emit_from(buf, ubase, nc_)

            cnt_s[0] = 0
            fire(c_start, chunk_a, sems[0])

            def pair_body(k, _):
                c = c_start + 2 * k

                @pl.when(2 * k + 1 < nch)
                def _():
                    fire(c + 1, chunk_b, sems[1])

                consume(c, chunk_a, sems[0])

                @pl.when(2 * k + 2 < nch)
                def _():
                    fire(c + 2, chunk_a, sems[0])

                @pl.when(2 * k + 1 < nch)
                def _():
                    consume(c + 1, chunk_b, sems[1])

                return 0

            lax.fori_loop(0, (nch + 1) // 2, pair_body, 0)

            nt = cscan_range(VMAIN, V)
            emit_from(tail_v, VMAIN, nt)

            nrows = cnt_s[0]

            def rdrain(k, _):
                pltpu.make_async_copy(tab_hbm.at[0, pl.ds(0, D)],
                                      rows_v.at[pl.ds(0, D)], sems[2]).wait()
                return 0

            lax.fori_loop(0, lax.min(nrows, RROWS), rdrain, 0)

        phase(user_hbm, utab_hbm, utail_hbm, ou_hbm)
        phase(item_hbm, itab_hbm, itail_hbm, oi_hbm)

    return extract(user, item, utab_t, itab_t, utail, itail)


def _join(user, item, uemb, iemb, ubias, ibias):
    B = user.shape[0]
    D = uemb.shape[0] // B
    bpw = B // NW

    mesh = plsc.VectorSubcoreMesh(
        core_axis_name="c", subcore_axis_name="s", num_cores=NC, num_subcores=NS
    )

    @functools.partial(
        pl.kernel,
        out_type=jax.ShapeDtypeStruct((B * D,), jnp.float32),
        mesh=mesh,
        scratch_types=[
            pltpu.VMEM((bpw,), jnp.int32),
            pltpu.VMEM((bpw,), jnp.int32),
            pltpu.VMEM((bpw * D,), jnp.float32),
            pltpu.VMEM((bpw * D,), jnp.float32),
            pltpu.VMEM((bpw,), jnp.float32),
            pltpu.VMEM((bpw,), jnp.float32),
            pltpu.VMEM((bpw * D,), jnp.float32),
            [pltpu.SemaphoreType.DMA] * 4,
        ],
    )
    def join(user_hbm, item_hbm, ue_hbm, ie_hbm, ub_hbm, ib_hbm, out_hbm,
             uidx_v, iidx_v, ue_v, ie_v, ub_v, ib_v, o_v, sems):
        wid = lax.axis_index("s") * NC + lax.axis_index("c")
        base = wid * bpw

        pltpu.sync_copy(user_hbm.at[pl.ds(base, bpw)], uidx_v)
        pltpu.sync_copy(item_hbm.at[pl.ds(base, bpw)], iidx_v)

        cub = pltpu.async_copy(ub_hbm.at[uidx_v], ub_v, sems[0])
        cib = pltpu.async_copy(ib_hbm.at[iidx_v], ib_v, sems[1])
        cue = pltpu.async_copy(ue_hbm.at[pl.ds(base * D, bpw * D)], ue_v,
                               sems[2])
        cie = pltpu.async_copy(ie_hbm.at[pl.ds(base * D, bpw * D)], ie_v,
                               sems[3])
        cub.wait()
        cib.wait()
        cue.wait()
        cie.wait()

        def blk(bi, _):
            b0 = bi * L
            ub16 = ub_v[pl.ds(b0, L)]
            ib16 = ib_v[pl.ds(b0, L)]
            for j in range(L):
                r = (b0 + j) * D
                ubb = jnp.full((L,), ub16[j])
                ibb = jnp.full((L,), ib16[j])
                for q in range(D // L):
                    sl = pl.ds(r + q * L, L)
                    o_v[sl] = (ue_v[sl] + ubb) * (ie_v[sl] + ibb)
            return 0

        lax.fori_loop(0, bpw // L, blk, 0)

        pltpu.sync_copy(o_v, out_hbm.at[pl.ds(base * D, bpw * D)])

    return join(user, item, uemb, iemb, ubias, ibias)


def kernel(user, item, user_table, item_table, user_bias, item_bias):
    B = user.shape[0]
    D = user_table.shape[1]
    V = user_table.shape[0]
    vmain = V - (V % 512)
    uemb, iemb = _extract(user, item, user_table.T, item_table.T,
                          user_table[vmain:].T, item_table[vmain:].T)
    out = _join(user, item, uemb, iemb, user_bias[:, 0], item_bias[:, 0])
    return out.reshape(B, D)


# speedup vs baseline: 1.0480x; 1.0057x over previous
import functools

import jax
import jax.numpy as jnp
from jax import lax
from jax.experimental import pallas as pl
from jax.experimental.pallas import tpu as pltpu
from jax.experimental.pallas import tpu_sc as plsc

NC = 2
NS = 16
L = 16
NW = NC * NS
CH = 512
CHLAST = 576
CSTRIDE = 640
MMAX = 768
RROWS = 32


def _extract(user, item, utab_t, itab_t, utail, itail):
    B = user.shape[0]
    D, V = utab_t.shape
    TAIL = V % CH
    VMAIN = V - TAIL
    nch_total = VMAIN // CH
    per_w = nch_total // NW
    extra = nch_total - per_w * NW

    mesh = plsc.VectorSubcoreMesh(
        core_axis_name="c", subcore_axis_name="s", num_cores=NC, num_subcores=NS
    )

    @functools.partial(
        pl.kernel,
        out_type=(jax.ShapeDtypeStruct((B * D,), jnp.float32),
                  jax.ShapeDtypeStruct((B * D,), jnp.float32)),
        mesh=mesh,
        compiler_params=pltpu.CompilerParams(needs_layout_passes=False),
        cost_estimate=pl.CostEstimate(flops=0, transcendentals=0,
                                      bytes_accessed=2 * 4 * D * V),
        scratch_types=[
            pltpu.VMEM((B,), jnp.int32),
            pltpu.VMEM((D, CH), jnp.float32),
            pltpu.VMEM((D, CH), jnp.float32),
            pltpu.VMEM((D, TAIL), jnp.float32),
            pltpu.VMEM((MMAX + L,), jnp.int32),
            pltpu.VMEM((MMAX + L,), jnp.int32),
            pltpu.VMEM((MMAX + L,), jnp.int32),
            pltpu.VMEM((MMAX + L,), jnp.int32),
            pltpu.VMEM((RROWS * 128,), jnp.float32),
            pltpu.SMEM((4,), jnp.int32),
            [pltpu.SemaphoreType.DMA] * 3,
        ],
    )
    def extract(user_hbm, item_hbm, utab_hbm, itab_hbm, utail_hbm, itail_hbm,
                ou_hbm, oi_hbm, idx_v, chunk_a, chunk_b, tail_v, mu_v, mb_v,
                cu_v, cb_v, rows_v, cnt_s, sems):
        wid = lax.axis_index("s") * NC + lax.axis_index("c")
        iota = lax.iota(jnp.int32, L)
        c_start = wid * per_w + lax.min(wid, extra)
        nch = per_w + jnp.where(wid < extra, 1, 0)

        def phase(idx_hbm, tab_hbm, tail_hbm, out_hbm):
            pltpu.sync_copy(idx_hbm, idx_v)
            pltpu.sync_copy(tail_hbm, tail_v)
            slab_lo = c_start * CH
            slab_hi = jnp.where(c_start + nch >= nch_total, V,
                                (c_start + nch) * CH)

            def scan(bi, ptr):
                for h in range(4):
                    b0 = bi * 4 * L + h * L
                    v16 = idx_v[pl.ds(b0, L)]
                    m = (v16 >= slab_lo) & (v16 < slab_hi)
                    plsc.store_compressed(mu_v.at[pl.ds(ptr, L)], v16, mask=m)
                    plsc.store_compressed(mb_v.at[pl.ds(ptr, L)], b0 + iota,
                                          mask=m)
                    ptr = ptr + plsc.all_reduce_population_count(m)[0]
                return ptr

            num = lax.fori_loop(0, B // (4 * L), scan, 0)
            mu_v[pl.ds(num, L)] = jnp.full((L,), -1, jnp.int32)

            def cscan_range(lo, hi):
                def cscan(bi, ptr):
                    v16 = mu_v[pl.ds(bi * L, L)]
                    b16 = mb_v[pl.ds(bi * L, L)]
                    m = (v16 >= lo) & (v16 < hi)
                    plsc.store_compressed(cu_v.at[pl.ds(ptr, L)], v16, mask=m)
                    plsc.store_compressed(cb_v.at[pl.ds(ptr, L)], b16, mask=m)
                    return ptr + plsc.all_reduce_population_count(m)[0]

                return lax.fori_loop(0, (num + L - 1) // L, cscan, 0)

            def emit_from(src_v, base_off, nc_):
                def emit(mi, rcnt):
                    ul = cu_v[pl.ds(mi, L)][0] - base_off
                    b = cb_v[pl.ds(mi, L)][0]
                    slot = rcnt & (RROWS - 1)

                    @pl.when(rcnt >= RROWS)
                    def _():
                        pltpu.make_async_copy(
                            tab_hbm.at[0, pl.ds(0, D)],
                            rows_v.at[pl.ds(0, D)], sems[2]).wait()

                    roff = pl.multiple_of(slot * 128, 128)
                    ulv = jnp.full((L,), ul, jnp.int32)
                    for q in range(D // L):
                        rows_v[pl.ds(roff + q * L, L)] = plsc.load_gather(
                            src_v, [q * L + iota, ulv])
                    pltpu.async_copy(rows_v.at[pl.ds(roff, D)],
                                     out_hbm.at[pl.ds(b * D, D)], sems[2])
                    return rcnt + 1

                cnt_s[0] = lax.fori_loop(0, nc_, emit, cnt_s[0])

            def fire(c, buf, sem):
                ubase = pl.multiple_of(c * CH, 128)
                pltpu.async_copy(tab_hbm.at[:, pl.ds(ubase, CH)], buf, sem)

            def consume(c, buf, sem):
                pltpu.make_async_copy(tab_hbm.at[:, pl.ds(0, CH)], buf,
                                      sem).wait()
                ubase = c * CH
                nc_ = cscan_range(ubase, ubase + CH)
                emit_from(buf, ubase, nc_)

            cnt_s[0] = 0
            fire(c_start, chunk_a, sems[0])

            def pair_body(k, _):
                c = c_start + 2 * k

                @pl.when(2 * k + 1 < nch)
                def _():
                    fire(c + 1, chunk_b, sems[1])

                consume(c, chunk_a, sems[0])

                @pl.when(2 * k + 2 < nch)
                def _():
                    fire(c + 2, chunk_a, sems[0])

                @pl.when(2 * k + 1 < nch)
                def _():
                    consume(c + 1, chunk_b, sems[1])

                return 0

            lax.fori_loop(0, (nch + 1) // 2, pair_body, 0)

            nt = cscan_range(VMAIN, V)
            emit_from(tail_v, VMAIN, nt)

            nrows = cnt_s[0]

            def rdrain(k, _):
                pltpu.make_async_copy(tab_hbm.at[0, pl.ds(0, D)],
                                      rows_v.at[pl.ds(0, D)], sems[2]).wait()
                return 0

            lax.fori_loop(0, lax.min(nrows, RROWS), rdrain, 0)

        phase(user_hbm, utab_hbm, utail_hbm, ou_hbm)
        phase(item_hbm, itab_hbm, itail_hbm, oi_hbm)

    return extract(user, item, utab_t, itab_t, utail, itail)


def _join(user, item, uemb, iemb, ubias, ibias):
    B = user.shape[0]
    D = uemb.shape[0] // B
    bpw = B // NW

    mesh = plsc.VectorSubcoreMesh(
        core_axis_name="c", subcore_axis_name="s", num_cores=NC, num_subcores=NS
    )

    @functools.partial(
        pl.kernel,
        out_type=jax.ShapeDtypeStruct((B * D,), jnp.float32),
        mesh=mesh,
        scratch_types=[
            pltpu.VMEM((bpw,), jnp.int32),
            pltpu.VMEM((bpw,), jnp.int32),
            pltpu.VMEM((bpw * D,), jnp.float32),
            pltpu.VMEM((bpw * D,), jnp.float32),
            pltpu.VMEM((bpw,), jnp.float32),
            pltpu.VMEM((bpw,), jnp.float32),
            pltpu.VMEM((bpw * D,), jnp.float32),
            [pltpu.SemaphoreType.DMA] * 4,
        ],
    )
    def join(user_hbm, item_hbm, ue_hbm, ie_hbm, ub_hbm, ib_hbm, out_hbm,
             uidx_v, iidx_v, ue_v, ie_v, ub_v, ib_v, o_v, sems):
        wid = lax.axis_index("s") * NC + lax.axis_index("c")
        base = wid * bpw

        pltpu.sync_copy(user_hbm.at[pl.ds(base, bpw)], uidx_v)
        pltpu.sync_copy(item_hbm.at[pl.ds(base, bpw)], iidx_v)

        cub = pltpu.async_copy(ub_hbm.at[uidx_v], ub_v, sems[0])
        cib = pltpu.async_copy(ib_hbm.at[iidx_v], ib_v, sems[1])
        cue = pltpu.async_copy(ue_hbm.at[pl.ds(base * D, bpw * D)], ue_v,
                               sems[2])
        cie = pltpu.async_copy(ie_hbm.at[pl.ds(base * D, bpw * D)], ie_v,
                               sems[3])
        cub.wait()
        cib.wait()
        cue.wait()
        cie.wait()

        def blk(bi, _):
            b0 = bi * L
            ub16 = ub_v[pl.ds(b0, L)]
            ib16 = ib_v[pl.ds(b0, L)]
            for j in range(L):
                r = (b0 + j) * D
                ubb = jnp.full((L,), ub16[j])
                ibb = jnp.full((L,), ib16[j])
                for q in range(D // L):
                    sl = pl.ds(r + q * L, L)
                    o_v[sl] = (ue_v[sl] + ubb) * (ie_v[sl] + ibb)
            return 0

        lax.fori_loop(0, bpw // L, blk, 0)

        pltpu.sync_copy(o_v, out_hbm.at[pl.ds(base * D, bpw * D)])

    return join(user, item, uemb, iemb, ubias, ibias)


def kernel(user, item, user_table, item_table, user_bias, item_bias):
    B = user.shape[0]
    D = user_table.shape[1]
    V = user_table.shape[0]
    vmain = V - (V % 512)
    uemb, iemb = _extract(user, item, user_table.T, item_table.T,
                          user_table[vmain:].T, item_table[vmain:].T)
    out = _join(user, item, uemb, iemb, user_bias[:, 0], item_bias[:, 0])
    return out.reshape(B, D)
